# Initial kernel scaffold; baseline (speedup 1.0000x reference)
import jax
import jax.numpy as jnp
from jax.experimental import pallas as pl


def kernel(embedding, edge_index, W1, b1, W2, b2):
    raise NotImplementedError("write your pallas kernel here")


# speedup vs baseline: 12.4363x; 12.4363x over previous
import functools

import jax
import jax.numpy as jnp
from jax import lax
from jax.experimental import pallas as pl
from jax.experimental.pallas import tpu as pltpu
from jax.experimental.pallas import tpu_sc as plsc

_NC = 2
_NS = 16
_NW = _NC * _NS
_C = 80
_BN = 1000


def _mesh():
    return plsc.VectorSubcoreMesh(
        core_axis_name="c", subcore_axis_name="s",
        num_cores=_NC, num_subcores=_NS)


def _pad_rows(n):
    return -(-n // (_NS * 128)) * (_NS * 128)


def _sc_degree(dst, n):
    (e,) = dst.shape
    e_per_w = e // _NW
    assert e_per_w * _NW == e and e_per_w % _C == 0
    nchunks = e_per_w // _C
    n_p = _pad_rows(n)
    rpt = n_p // _NS
    zr = 128
    nz = rpt // zr
    assert nz * zr == rpt
    f = 128

    def body(dst_hbm, out_hbm, idx_v, ones_v, zbuf_v, acc_sh):
        c = lax.axis_index("c")
        s = lax.axis_index("s")
        wid = s * _NC + c

        def fill_ones(i, _):
            for kk in range(f // 16):
                ones_v[i, pl.ds(kk * 16, 16)] = jnp.full((16,), 1.0,
                                                         jnp.float32)
            return 0
        lax.fori_loop(0, _C, fill_ones, 0)

        def fill_zero(i, _):
            for kk in range(f // 16):
                zbuf_v[i, pl.ds(kk * 16, 16)] = jnp.zeros((16,), jnp.float32)
            return 0
        lax.fori_loop(0, zr, fill_zero, 0)

        def zero_acc(k, _):
            pltpu.sync_copy(zbuf_v, acc_sh.at[pl.ds(s * rpt + k * zr, zr)])
            return 0
        lax.fori_loop(0, nz, zero_acc, 0)
        plsc.subcore_barrier()

        base = wid * e_per_w

        def chunk(j, _):
            pltpu.sync_copy(dst_hbm.at[pl.ds(base + j * _C, _C)], idx_v)
            pltpu.sync_copy(ones_v, acc_sh.at[idx_v], add=True)
            return 0
        lax.fori_loop(0, nchunks, chunk, 0)
        plsc.subcore_barrier()

        pltpu.sync_copy(acc_sh.at[pl.ds(s * rpt, rpt)],
                        out_hbm.at[c, pl.ds(s * rpt, rpt)])

    k = pl.kernel(
        body,
        out_type=jax.ShapeDtypeStruct((_NC, n_p, f), jnp.float32),
        mesh=_mesh(),
        scratch_types=[
            pltpu.VMEM((_C,), jnp.int32),
            pltpu.VMEM((_C, f), jnp.float32),
            pltpu.VMEM((zr, f), jnp.float32),
            pltpu.VMEM_SHARED((n_p, f), jnp.float32),
        ])
    return k(dst)


def _sc_scatter_sum(g, src, dst):
    n, f = g.shape
    (e,) = src.shape
    e_per_w = e // _NW
    assert e_per_w * _NW == e and e_per_w % _C == 0
    nchunks = e_per_w // _C
    n_p = _pad_rows(n)
    rpt = n_p // _NS
    zr = 128
    nz = rpt // zr
    assert nz * zr == rpt
    nvz = f // 16
    assert nvz * 16 == f

    def body(g_hbm, src_hbm, dst_hbm, out_hbm,
             si_v, di_v, rows_v, zbuf_v, acc_sh, sem):
        c = lax.axis_index("c")
        s = lax.axis_index("s")
        wid = s * _NC + c

        def fill_zero(i, _):
            for kk in range(nvz):
                zbuf_v[i, pl.ds(kk * 16, 16)] = jnp.zeros((16,), jnp.float32)
            return 0
        lax.fori_loop(0, zr, fill_zero, 0)

        def zero_acc(k2, _):
            pltpu.sync_copy(zbuf_v, acc_sh.at[pl.ds(s * rpt + k2 * zr, zr)])
            return 0
        lax.fori_loop(0, nz, zero_acc, 0)
        plsc.subcore_barrier()

        base = wid * e_per_w

        def chunk(j, _):
            off = base + j * _C
            pltpu.sync_copy(src_hbm.at[pl.ds(off, _C)], si_v)
            pltpu.sync_copy(dst_hbm.at[pl.ds(off, _C)], di_v)
            pltpu.async_copy(g_hbm.at[si_v], rows_v, sem).wait()
            pltpu.sync_copy(rows_v, acc_sh.at[di_v], add=True)
            return 0
        lax.fori_loop(0, nchunks, chunk, 0)
        plsc.subcore_barrier()

        pltpu.sync_copy(acc_sh.at[pl.ds(s * rpt, rpt)],
                        out_hbm.at[c, pl.ds(s * rpt, rpt)])

    k = pl.kernel(
        body,
        out_type=jax.ShapeDtypeStruct((_NC, n_p, f), jnp.float32),
        mesh=_mesh(),
        scratch_types=[
            pltpu.VMEM((_C,), jnp.int32),
            pltpu.VMEM((_C,), jnp.int32),
            pltpu.VMEM((_C, f), jnp.float32),
            pltpu.VMEM((zr, f), jnp.float32),
            pltpu.VMEM_SHARED((n_p, f), jnp.float32),
            pltpu.SemaphoreType.DMA,
        ])
    return k(g, src, dst)


def _dis_block(degp_ref):
    deg = degp_ref[0, :, 0:1] + degp_ref[1, :, 0:1] + 1.0
    return lax.rsqrt(deg)


def _tc_scale_matmul(degp, x, w):
    n, d = x.shape
    h = w.shape[1]
    hp = max(h, 128)

    def body(degp_ref, x_ref, w_ref, o_ref):
        dis = _dis_block(degp_ref)
        mm = jnp.dot(x_ref[...], w_ref[...],
                     preferred_element_type=jnp.float32) * dis
        if hp > h:
            mm = jnp.concatenate(
                [mm, jnp.zeros((mm.shape[0], hp - h), jnp.float32)], axis=1)
        o_ref[...] = mm

    return pl.pallas_call(
        body,
        grid=(n // _BN,),
        in_specs=[
            pl.BlockSpec((2, _BN, 128), lambda i: (0, i, 0)),
            pl.BlockSpec((_BN, d), lambda i: (i, 0)),
            pl.BlockSpec((d, h), lambda i: (0, 0)),
        ],
        out_specs=pl.BlockSpec((_BN, hp), lambda i: (i, 0)),
        out_shape=jax.ShapeDtypeStruct((n, hp), jnp.float32),
    )(degp, x, w)


def _tc_mid(degp, p, g, w, b):
    n, hp = g.shape
    h = w.shape[0]
    d2 = w.shape[1]

    def body(degp_ref, p_ref, g_ref, w_ref, b_ref, o_ref):
        dis = _dis_block(degp_ref)
        acc = p_ref[0, :, 0:h] + p_ref[1, :, 0:h] + g_ref[:, 0:h]
        y = jnp.maximum(acc * dis + b_ref[...], 0.0)
        o_ref[...] = jnp.dot(y, w_ref[...],
                             preferred_element_type=jnp.float32) * dis

    return pl.pallas_call(
        body,
        grid=(n // _BN,),
        in_specs=[
            pl.BlockSpec((2, _BN, 128), lambda i: (0, i, 0)),
            pl.BlockSpec((2, _BN, hp), lambda i: (0, i, 0)),
            pl.BlockSpec((_BN, hp), lambda i: (i, 0)),
            pl.BlockSpec((h, d2), lambda i: (0, 0)),
            pl.BlockSpec((1, h), lambda i: (0, 0)),
        ],
        out_specs=pl.BlockSpec((_BN, d2), lambda i: (i, 0)),
        out_shape=jax.ShapeDtypeStruct((n, d2), jnp.float32),
    )(degp, p, g, w, b)


def _tc_out(degp, p, g, b):
    n, d2 = g.shape

    def body(degp_ref, p_ref, g_ref, b_ref, o_ref):
        dis = _dis_block(degp_ref)
        acc = p_ref[0] + p_ref[1] + g_ref[...]
        o_ref[...] = jnp.maximum(acc * dis + b_ref[...], 0.0)

    return pl.pallas_call(
        body,
        grid=(n // _BN,),
        in_specs=[
            pl.BlockSpec((2, _BN, 128), lambda i: (0, i, 0)),
            pl.BlockSpec((2, _BN, d2), lambda i: (0, i, 0)),
            pl.BlockSpec((_BN, d2), lambda i: (i, 0)),
            pl.BlockSpec((1, d2), lambda i: (0, 0)),
        ],
        out_specs=pl.BlockSpec((_BN, d2), lambda i: (i, 0)),
        out_shape=jax.ShapeDtypeStruct((n, d2), jnp.float32),
    )(degp, p, g, b)


def kernel(embedding, edge_index, W1, b1, W2, b2):
    x = embedding
    src = edge_index[0]
    dst = edge_index[1]
    n = x.shape[0]

    degp = _sc_degree(dst, n)
    g1 = _tc_scale_matmul(degp, x, W1)
    p1 = _sc_scatter_sum(g1, src, dst)
    g2 = _tc_mid(degp, p1, g1, W2, b1.reshape(1, -1))
    p2 = _sc_scatter_sum(g2, src, dst)
    out = _tc_out(degp, p2, g2, b2.reshape(1, -1))
    return out


# speedup vs baseline: 26.3909x; 2.1221x over previous
import jax
import jax.numpy as jnp
from jax import lax
from jax.experimental import pallas as pl
from jax.experimental.pallas import tpu as pltpu
from jax.experimental.pallas import tpu_sc as plsc

_NC = 2
_NS = 16
_NW = _NC * _NS
_C = 80
_BN = 1000


def _mesh():
    return plsc.VectorSubcoreMesh(
        core_axis_name="c", subcore_axis_name="s",
        num_cores=_NC, num_subcores=_NS)


def _pad_rows(n):
    return -(-n // (_NS * 128)) * (_NS * 128)


def _sc_degree(dst3, n, f):
    nw, nchunks, c_ = dst3.shape
    assert nw == _NW and c_ == _C
    n_p = _pad_rows(n)
    rpt = n_p // _NS
    zr = 16
    nz = rpt // zr
    assert nz * zr == rpt

    def body(dst_hbm, out_hbm, di_all, ones_v, zbuf_v, acc_sh):
        c = lax.axis_index("c")
        s = lax.axis_index("s")
        wid = s * _NC + c

        pltpu.sync_copy(dst_hbm.at[wid], di_all)

        def fill_ones(i, _):
            for kk in range(f // 16):
                ones_v[i, pl.ds(kk * 16, 16)] = jnp.full((16,), 1.0,
                                                         jnp.float32)
            return 0
        lax.fori_loop(0, _C, fill_ones, 0)

        def fill_zero(i, _):
            for kk in range(f // 16):
                zbuf_v[i, pl.ds(kk * 16, 16)] = jnp.zeros((16,), jnp.float32)
            return 0
        lax.fori_loop(0, zr, fill_zero, 0)

        def zero_acc(k, _):
            pltpu.sync_copy(zbuf_v, acc_sh.at[pl.ds(s * rpt + k * zr, zr)])
            return 0
        lax.fori_loop(0, nz, zero_acc, 0)
        plsc.subcore_barrier()

        def chunk(j, _):
            pltpu.sync_copy(ones_v, acc_sh.at[di_all.at[j]], add=True)
            return 0
        lax.fori_loop(0, nchunks, chunk, 0)
        plsc.subcore_barrier()

        pltpu.sync_copy(acc_sh.at[pl.ds(s * rpt, rpt)],
                        out_hbm.at[c, pl.ds(s * rpt, rpt)])

    k = pl.kernel(
        body,
        out_type=jax.ShapeDtypeStruct((_NC, n_p, f), jnp.float32),
        mesh=_mesh(),
        scratch_types=[
            pltpu.VMEM((nchunks, _C), jnp.int32),
            pltpu.VMEM((_C, f), jnp.float32),
            pltpu.VMEM((zr, f), jnp.float32),
            pltpu.VMEM_SHARED((n_p, f), jnp.float32),
        ])
    return k(dst3)


def _sc_scatter_sum(g, packed3):
    n, f = g.shape
    nw, nchunks, c_ = packed3.shape
    assert nw == _NW and c_ == _C and nchunks % 2 == 1
    n_p = _pad_rows(n)
    rpt = n_p // _NS
    zr = 16
    nz = rpt // zr
    assert nz * zr == rpt
    npairs = (nchunks - 1) // 2
    mask14 = jnp.int32((1 << 14) - 1)

    def body(g_hbm, pk_hbm, out_hbm,
             pk_all, si0, di0, si1, di1, rows0, rows1, zbuf_v, acc_sh,
             sem0, sem1):
        c = lax.axis_index("c")
        s = lax.axis_index("s")
        wid = s * _NC + c

        pltpu.sync_copy(pk_hbm.at[wid], pk_all)

        def fill_zero(i, _):
            for kk in range(f // 16):
                zbuf_v[i, pl.ds(kk * 16, 16)] = jnp.zeros((16,), jnp.float32)
            return 0
        lax.fori_loop(0, zr, fill_zero, 0)

        def zero_acc(k2, _):
            pltpu.sync_copy(zbuf_v, acc_sh.at[pl.ds(s * rpt + k2 * zr, zr)])
            return 0
        lax.fori_loop(0, nz, zero_acc, 0)
        plsc.subcore_barrier()

        def unpack(j, si_v, di_v):
            for kk in range(_C // 16):
                pk = pk_all[j, pl.ds(kk * 16, 16)]
                si_v[pl.ds(kk * 16, 16)] = pk & mask14
                di_v[pl.ds(kk * 16, 16)] = lax.shift_right_logical(pk, 14)

        unpack(0, si0, di0)
        pltpu.async_copy(g_hbm.at[si0], rows0, sem0)

        def pair(jj, _):
            j0 = jj * 2
            unpack(j0 + 1, si1, di1)
            pltpu.async_copy(g_hbm.at[si1], rows1, sem1)
            pltpu.make_async_copy(g_hbm.at[si0], rows0, sem0).wait()
            pltpu.sync_copy(rows0, acc_sh.at[di0], add=True)
            unpack(j0 + 2, si0, di0)
            pltpu.async_copy(g_hbm.at[si0], rows0, sem0)
            pltpu.make_async_copy(g_hbm.at[si1], rows1, sem1).wait()
            pltpu.sync_copy(rows1, acc_sh.at[di1], add=True)
            return 0
        lax.fori_loop(0, npairs, pair, 0)

        pltpu.make_async_copy(g_hbm.at[si0], rows0, sem0).wait()
        pltpu.sync_copy(rows0, acc_sh.at[di0], add=True)
        plsc.subcore_barrier()

        pltpu.sync_copy(acc_sh.at[pl.ds(s * rpt, rpt)],
                        out_hbm.at[c, pl.ds(s * rpt, rpt)])

    k = pl.kernel(
        body,
        out_type=jax.ShapeDtypeStruct((_NC, n_p, f), jnp.float32),
        mesh=_mesh(),
        scratch_types=[
            pltpu.VMEM((nchunks, _C), jnp.int32),
            pltpu.VMEM((_C,), jnp.int32),
            pltpu.VMEM((_C,), jnp.int32),
            pltpu.VMEM((_C,), jnp.int32),
            pltpu.VMEM((_C,), jnp.int32),
            pltpu.VMEM((_C, f), jnp.float32),
            pltpu.VMEM((_C, f), jnp.float32),
            pltpu.VMEM((zr, f), jnp.float32),
            pltpu.VMEM_SHARED((n_p, f), jnp.float32),
            pltpu.SemaphoreType.DMA,
            pltpu.SemaphoreType.DMA,
        ])
    return k(g, packed3)


def _dis_block(degp_ref):
    deg = degp_ref[0, :, 0:1] + degp_ref[1, :, 0:1] + 1.0
    return lax.rsqrt(deg)


def _tc_scale_matmul(degp, x, w, hp):
    n, d = x.shape
    h = w.shape[1]
    df = degp.shape[2]

    def body(degp_ref, x_ref, w_ref, o_ref):
        dis = _dis_block(degp_ref)
        mm = jnp.dot(x_ref[...], w_ref[...],
                     preferred_element_type=jnp.float32) * dis
        if hp > h:
            mm = jnp.concatenate(
                [mm, jnp.zeros((mm.shape[0], hp - h), jnp.float32)], axis=1)
        o_ref[...] = mm

    return pl.pallas_call(
        body,
        grid=(n // _BN,),
        in_specs=[
            pl.BlockSpec((2, _BN, df), lambda i: (0, i, 0)),
            pl.BlockSpec((_BN, d), lambda i: (i, 0)),
            pl.BlockSpec((d, h), lambda i: (0, 0)),
        ],
        out_specs=pl.BlockSpec((_BN, hp), lambda i: (i, 0)),
        out_shape=jax.ShapeDtypeStruct((n, hp), jnp.float32),
    )(degp, x, w)


def _tc_mid(degp, p, g, w, b):
    n, hp = g.shape
    h = w.shape[0]
    d2 = w.shape[1]
    df = degp.shape[2]

    def body(degp_ref, p_ref, g_ref, w_ref, b_ref, o_ref):
        dis = _dis_block(degp_ref)
        acc = p_ref[0, :, 0:h] + p_ref[1, :, 0:h] + g_ref[:, 0:h]
        y = jnp.maximum(acc * dis + b_ref[...], 0.0)
        o_ref[...] = jnp.dot(y, w_ref[...],
                             preferred_element_type=jnp.float32) * dis

    return pl.pallas_call(
        body,
        grid=(n // _BN,),
        in_specs=[
            pl.BlockSpec((2, _BN, df), lambda i: (0, i, 0)),
            pl.BlockSpec((2, _BN, hp), lambda i: (0, i, 0)),
            pl.BlockSpec((_BN, hp), lambda i: (i, 0)),
            pl.BlockSpec((h, d2), lambda i: (0, 0)),
            pl.BlockSpec((1, h), lambda i: (0, 0)),
        ],
        out_specs=pl.BlockSpec((_BN, d2), lambda i: (i, 0)),
        out_shape=jax.ShapeDtypeStruct((n, d2), jnp.float32),
    )(degp, p, g, w, b)


def _tc_out(degp, p, g, b):
    n, d2 = g.shape
    df = degp.shape[2]

    def body(degp_ref, p_ref, g_ref, b_ref, o_ref):
        dis = _dis_block(degp_ref)
        acc = p_ref[0] + p_ref[1] + g_ref[...]
        o_ref[...] = jnp.maximum(acc * dis + b_ref[...], 0.0)

    return pl.pallas_call(
        body,
        grid=(n // _BN,),
        in_specs=[
            pl.BlockSpec((2, _BN, df), lambda i: (0, i, 0)),
            pl.BlockSpec((2, _BN, d2), lambda i: (0, i, 0)),
            pl.BlockSpec((_BN, d2), lambda i: (i, 0)),
            pl.BlockSpec((1, d2), lambda i: (0, 0)),
        ],
        out_specs=pl.BlockSpec((_BN, d2), lambda i: (i, 0)),
        out_shape=jax.ShapeDtypeStruct((n, d2), jnp.float32),
    )(degp, p, g, b)


def kernel(embedding, edge_index, W1, b1, W2, b2):
    x = embedding
    n = x.shape[0]
    e = edge_index.shape[1]
    e_per_w = e // _NW
    assert e_per_w * _NW == e and e_per_w % _C == 0
    nchunks = e_per_w // _C
    dst3 = edge_index[1].reshape(_NW, nchunks, _C)
    packed3 = (edge_index[0] | (edge_index[1] << 14)).reshape(
        _NW, nchunks, _C)

    degp = _sc_degree(dst3, n, 128)
    g1 = _tc_scale_matmul(degp, x, W1, 128)
    p1 = _sc_scatter_sum(g1, packed3)
    g2 = _tc_mid(degp, p1, g1, W2, b1.reshape(1, -1))
    p2 = _sc_scatter_sum(g2, packed3)
    out = _tc_out(degp, p2, g2, b2.reshape(1, -1))
    return out
